# baseline (speedup 1.0000x reference)
import functools

import jax
import jax.numpy as jnp
from jax import lax
from jax.experimental import pallas as pl
from jax.experimental.pallas import tpu as pltpu
from jax.experimental.pallas import tpu_sc as plsc

N = 4096
F_IN = 512
D_EMB = 512
OUT_F = 128
E_EDGES = 65536
AUG = 640
AGG_W = 528
BLK_A = 256
BLK_B = 128
BIG = 1.0e30
PREC = lax.Precision.HIGHEST

SC_NC = 2
SC_NS = 16
NW = SC_NC * SC_NS
ROWS_PER_TILE = N // NW
CAP = 2048
CHUNK = 2048
GATHER_B = N // NW


def _dot(a, b, dims, prec=PREC):
    return lax.dot_general(a, b, (dims, ((), ())),
                           preferred_element_type=jnp.float32, precision=prec)


def _embed_body(feats_ref, wp_ref, emb_ref, aug_ref, sq_ref, esq_ref):
    f = feats_ref[...]
    e = jnp.maximum(_dot(f, wp_ref[...], ((1,), (0,)),
                         prec=lax.Precision.DEFAULT), 0.0)
    emb_ref[...] = e
    ones_col = (lax.broadcasted_iota(jnp.int32, (BLK_A, AUG - D_EMB), 1) == 0)
    aug_ref[...] = jnp.concatenate([e, ones_col.astype(jnp.float32)], axis=1)
    sq_ref[...] = jnp.sum(f * f, axis=1, keepdims=True)
    esq_ref[...] = jnp.sum(e * e, axis=1, keepdims=True)


def _embed_call(feats, w_patch):
    return pl.pallas_call(
        _embed_body,
        grid=(N // BLK_A,),
        in_specs=[
            pl.BlockSpec((BLK_A, F_IN), lambda i: (i, 0)),
            pl.BlockSpec((F_IN, D_EMB), lambda i: (0, 0)),
        ],
        out_specs=[
            pl.BlockSpec((BLK_A, D_EMB), lambda i: (i, 0)),
            pl.BlockSpec((BLK_A, AUG), lambda i: (i, 0)),
            pl.BlockSpec((BLK_A, 1), lambda i: (i, 0)),
            pl.BlockSpec((BLK_A, 1), lambda i: (i, 0)),
        ],
        out_shape=[
            jax.ShapeDtypeStruct((N, D_EMB), jnp.float32),
            jax.ShapeDtypeStruct((N, AUG), jnp.float32),
            jax.ShapeDtypeStruct((N, 1), jnp.float32),
            jax.ShapeDtypeStruct((N, 1), jnp.float32),
        ],
    )(feats, w_patch)


def _mine_body(fb_ref, fall_ref, eb_ref, eall_ref, sqb_ref, sqr_ref, esqr_ref,
               pos_ref, neg_ref):
    i = pl.program_id(0)
    g = _dot(fb_ref[...], fall_ref[...], ((1,), (1,)),
             prec=lax.Precision.DEFAULT)
    d2 = sqb_ref[...] + sqr_ref[...] - 2.0 * g
    iota_i = lax.broadcasted_iota(jnp.int32, (BLK_B, N), 1)
    iota_f = iota_i.astype(jnp.float32)
    rowid = i * BLK_B + lax.broadcasted_iota(jnp.int32, (BLK_B, N), 0)

    dpos = jnp.where(iota_i == rowid, BIG, d2)
    pos_i = []
    for _ in range(5):
        m = jnp.min(dpos, axis=1, keepdims=True)
        idx = jnp.min(jnp.where(dpos == m, iota_f, BIG), axis=1, keepdims=True)
        pos_i.append(idx)
        dpos = jnp.where(iota_f == idx, BIG, dpos)

    dneg = d2
    neg_i = [None] * 5
    for k in range(4, -1, -1):
        m = jnp.max(dneg, axis=1, keepdims=True)
        idx = jnp.max(jnp.where(dneg == m, iota_f, -1.0), axis=1, keepdims=True)
        neg_i[k] = idx
        dneg = jnp.where(iota_f == idx, -BIG, dneg)

    gemb = _dot(eb_ref[...], eall_ref[...], ((1,), (1,)))
    t = esqr_ref[...] - 2.0 * gemb

    def pick(vals):
        return [jnp.sum(jnp.where(iota_f == c, t, 0.0), axis=1, keepdims=True)
                for c in vals]

    d_pos = pick(pos_i)
    d_neg = pick(neg_i)

    best_v, best_i = d_pos[0], pos_i[0]
    for k in range(1, 5):
        upd = d_pos[k] > best_v
        best_v = jnp.where(upd, d_pos[k], best_v)
        best_i = jnp.where(upd, pos_i[k], best_i)
    pos_ref[...] = best_i.astype(jnp.int32)

    best_v, best_i = d_neg[0], neg_i[0]
    for k in range(1, 5):
        upd = d_neg[k] < best_v
        best_v = jnp.where(upd, d_neg[k], best_v)
        best_i = jnp.where(upd, neg_i[k], best_i)
    neg_ref[...] = best_i.astype(jnp.int32)


def _mine_call(feats, emb, sq, sq_c, esq_c):
    return pl.pallas_call(
        _mine_body,
        grid=(N // BLK_B,),
        in_specs=[
            pl.BlockSpec((BLK_B, F_IN), lambda i: (i, 0)),
            pl.BlockSpec((N, F_IN), lambda i: (0, 0)),
            pl.BlockSpec((BLK_B, D_EMB), lambda i: (i, 0)),
            pl.BlockSpec((N, D_EMB), lambda i: (0, 0)),
            pl.BlockSpec((BLK_B, 1), lambda i: (i, 0)),
            pl.BlockSpec((1, N), lambda i: (0, 0)),
            pl.BlockSpec((1, N), lambda i: (0, 0)),
        ],
        out_specs=[
            pl.BlockSpec((BLK_B, 1), lambda i: (i, 0)),
            pl.BlockSpec((BLK_B, 1), lambda i: (i, 0)),
        ],
        out_shape=[
            jax.ShapeDtypeStruct((N, 1), jnp.int32),
            jax.ShapeDtypeStruct((N, 1), jnp.int32),
        ],
    )(feats, feats, emb, emb, sq, sq_c, esq_c)


def _mesh_body(agg_ref, emb_ref, wm_ref, wc_ref, mg_ref, log_ref):
    aug = agg_ref[...]
    lane = lax.broadcasted_iota(jnp.int32, (N, AGG_W), 1)
    deg = jnp.sum(jnp.where(lane == D_EMB, aug, 0.0), axis=1, keepdims=True)
    agg = aug[:, :D_EMB] / jnp.maximum(deg, 1.0)
    h = jnp.maximum(_dot(agg, wm_ref[...], ((1,), (0,))), 0.0) + emb_ref[...]
    mg = jnp.sum(h, axis=0, keepdims=True) * (1.0 / N)
    mg_ref[...] = mg
    log_ref[...] = _dot(mg, wc_ref[...], ((1,), (0,)))


def _mesh_call(agg_aug, emb, w_mesh, w_cls):
    return pl.pallas_call(
        _mesh_body,
        out_shape=[
            jax.ShapeDtypeStruct((1, D_EMB), jnp.float32),
            jax.ShapeDtypeStruct((1, OUT_F), jnp.float32),
        ],
    )(agg_aug, emb, w_mesh, w_cls)


PASS_ROWS = 64


def _segsum_body(src_hbm, dst_hbm, aug_hbm, out_hbm,
                 srcb, dstb, slist, dlist, agg, stage, sem):
    wid = lax.axis_index("s") * SC_NC + lax.axis_index("c")
    iota16 = lax.iota(jnp.int32, 16)
    zvec16 = jnp.zeros((16,), jnp.int32)

    for p in range(ROWS_PER_TILE // PASS_ROWS):
        lo = wid * ROWS_PER_TILE + p * PASS_ROWS

        def zero_row(r, carry):
            def zero_col(c, carry2):
                agg[r, pl.ds(c * 16, 16)] = jnp.zeros((16,), jnp.float32)
                return carry2
            return lax.fori_loop(0, AGG_W // 16, zero_col, carry)
        lax.fori_loop(0, PASS_ROWS + 4, zero_row, 0)

        def scan_chunk(ci, off):
            pltpu.sync_copy(src_hbm.at[pl.ds(ci * CHUNK, CHUNK)], srcb)
            pltpu.sync_copy(dst_hbm.at[pl.ds(ci * CHUNK, CHUNK)], dstb)

            def scan_vec(vi, off2):
                d = dstb[pl.ds(vi * 16, 16)]
                s = srcb[pl.ds(vi * 16, 16)]
                msk = (d >= lo) & (d < lo + PASS_ROWS)
                mi = jnp.where(msk, 1, 0)
                cum = plsc.cumsum(mi)
                pos = jnp.where(msk, off2 + cum - 1, CAP + 8)
                plsc.store_scatter(slist, [pos], s)
                plsc.store_scatter(dlist, [pos >> 4, pos & 15], d - lo)
                return off2 + cum[15]
            return lax.fori_loop(0, CHUNK // 16, scan_vec, off)
        off = lax.fori_loop(0, E_EDGES // CHUNK, scan_chunk, jnp.int32(0))

        pos_pad = off + iota16
        plsc.store_scatter(slist, [pos_pad], zvec16)
        plsc.store_scatter(dlist, [pos_pad >> 4, pos_pad & 15],
                           jnp.full((16,), PASS_ROWS, jnp.int32))

        nb = (off + 15) >> 4

        def start_gather(b):
            sidx = slist[pl.ds(b * 16, 16)]
            pltpu.async_copy(aug_hbm.at[sidx], stage.at[b % 3], sem)

        @pl.when(nb > 0)
        def _():
            start_gather(jnp.int32(0))

        @pl.when(nb > 1)
        def _():
            start_gather(jnp.int32(1))

        def proc(bi, carry):
            cur = bi % 3
            pltpu.make_async_copy(aug_hbm.at[pl.ds(0, 16)],
                                  stage.at[cur], sem).wait()

            @pl.when(bi + 2 < nb)
            def _():
                start_gather(bi + 2)
            dl = dlist[bi]
            for r in range(16):
                d_r = dl[r]
                for c in range(AGG_W // 16):
                    sl = pl.ds(c * 16, 16)
                    agg[d_r, sl] = agg[d_r, sl] + stage[cur, r, sl]
            return carry
        lax.fori_loop(0, nb, proc, 0)

        pltpu.sync_copy(agg.at[pl.ds(0, PASS_ROWS)],
                        out_hbm.at[pl.ds(lo, PASS_ROWS)])


def _segsum_call(src, dst, emb_aug):
    mesh = plsc.VectorSubcoreMesh(core_axis_name="c", subcore_axis_name="s")
    k = functools.partial(
        pl.kernel,
        mesh=mesh,
        compiler_params=pltpu.CompilerParams(needs_layout_passes=False),
        out_type=jax.ShapeDtypeStruct((N, AGG_W), jnp.float32),
        scratch_types=[
            pltpu.VMEM((CHUNK,), jnp.int32),
            pltpu.VMEM((CHUNK,), jnp.int32),
            pltpu.VMEM((CAP + 16,), jnp.int32),
            pltpu.VMEM((CAP // 16 + 2, 16), jnp.int32),
            pltpu.VMEM((PASS_ROWS + 4, AGG_W), jnp.float32),
            pltpu.VMEM((3, 16, AUG), jnp.float32),
            pltpu.SemaphoreType.DMA,
        ],
    )(_segsum_body)
    return k(src, dst, emb_aug)


def _gather_body(emb_hbm, pos_hbm, neg_hbm, pos_out, neg_out,
                 idx_v, rows_v, sem):
    wid = lax.axis_index("s") * SC_NC + lax.axis_index("c")
    base = wid * GATHER_B
    pltpu.sync_copy(pos_hbm.at[pl.ds(base, GATHER_B)], idx_v)
    pltpu.async_copy(emb_hbm.at[idx_v], rows_v, sem).wait()
    pltpu.sync_copy(rows_v, pos_out.at[pl.ds(base, GATHER_B)])
    pltpu.sync_copy(neg_hbm.at[pl.ds(base, GATHER_B)], idx_v)
    pltpu.async_copy(emb_hbm.at[idx_v], rows_v, sem).wait()
    pltpu.sync_copy(rows_v, neg_out.at[pl.ds(base, GATHER_B)])


def _gather_call(emb, pos_idx, neg_idx):
    mesh = plsc.VectorSubcoreMesh(core_axis_name="c", subcore_axis_name="s")
    k = functools.partial(
        pl.kernel,
        mesh=mesh,
        compiler_params=pltpu.CompilerParams(needs_layout_passes=False),
        out_type=(
            jax.ShapeDtypeStruct((N, D_EMB), jnp.float32),
            jax.ShapeDtypeStruct((N, D_EMB), jnp.float32),
        ),
        scratch_types=[
            pltpu.VMEM((GATHER_B,), jnp.int32),
            pltpu.VMEM((GATHER_B, D_EMB), jnp.float32),
            pltpu.SemaphoreType.DMA,
        ],
    )(_gather_body)
    return k(emb, pos_idx, neg_idx)


def kernel(feats, edge_index, W_patch, W_mesh, W_cls):
    emb, emb_aug, sq, esq = _embed_call(feats, W_patch)
    sq_c = sq.reshape(1, N)
    esq_c = esq.reshape(1, N)
    agg_aug = _segsum_call(edge_index[0], edge_index[1], emb_aug)
    pos_i, neg_i = _mine_call(feats, emb, sq, sq_c, esq_c)
    mg, logits = _mesh_call(agg_aug, emb, W_mesh, W_cls)
    pos_emb, neg_emb = _gather_call(emb, pos_i.reshape(N), neg_i.reshape(N))
    return (logits.reshape(OUT_F), mg.reshape(D_EMB), emb, pos_emb, neg_emb)


# speedup vs baseline: 1.2660x; 1.2660x over previous
import functools

import jax
import jax.numpy as jnp
from jax import lax
from jax.experimental import pallas as pl
from jax.experimental.pallas import tpu as pltpu
from jax.experimental.pallas import tpu_sc as plsc

N = 4096
F_IN = 512
D_EMB = 512
OUT_F = 128
E_EDGES = 65536
AUG = 640
AGG_W = 528
BLK_A = 256
BLK_B = 128
BIG = 1.0e30
PREC = lax.Precision.HIGHEST

SC_NC = 2
SC_NS = 16
NW = SC_NC * SC_NS
ROWS_PER_TILE = N // NW
CAP = 2048
CHUNK = 2048
GATHER_B = N // NW


def _dot(a, b, dims, prec=PREC):
    return lax.dot_general(a, b, (dims, ((), ())),
                           preferred_element_type=jnp.float32, precision=prec)


def _embed_body(feats_ref, wp_ref, emb_ref, aug_ref, sq_ref, esq_ref):
    f = feats_ref[...]
    e = jnp.maximum(_dot(f, wp_ref[...], ((1,), (0,)),
                         prec=lax.Precision.DEFAULT), 0.0)
    emb_ref[...] = e
    ones_col = (lax.broadcasted_iota(jnp.int32, (BLK_A, AUG - D_EMB), 1) == 0)
    aug_ref[...] = jnp.concatenate([e, ones_col.astype(jnp.float32)], axis=1)
    sq_ref[...] = jnp.sum(f * f, axis=1, keepdims=True)
    esq_ref[...] = jnp.sum(e * e, axis=1, keepdims=True)


def _embed_call(feats, w_patch):
    return pl.pallas_call(
        _embed_body,
        grid=(N // BLK_A,),
        in_specs=[
            pl.BlockSpec((BLK_A, F_IN), lambda i: (i, 0)),
            pl.BlockSpec((F_IN, D_EMB), lambda i: (0, 0)),
        ],
        out_specs=[
            pl.BlockSpec((BLK_A, D_EMB), lambda i: (i, 0)),
            pl.BlockSpec((BLK_A, AUG), lambda i: (i, 0)),
            pl.BlockSpec((BLK_A, 1), lambda i: (i, 0)),
            pl.BlockSpec((BLK_A, 1), lambda i: (i, 0)),
        ],
        out_shape=[
            jax.ShapeDtypeStruct((N, D_EMB), jnp.float32),
            jax.ShapeDtypeStruct((N, AUG), jnp.float32),
            jax.ShapeDtypeStruct((N, 1), jnp.float32),
            jax.ShapeDtypeStruct((N, 1), jnp.float32),
        ],
    )(feats, w_patch)


def _mine_body(fb_ref, fall_ref, eb_ref, eall_ref, sqb_ref, sqr_ref, esqr_ref,
               pos_ref, neg_ref):
    i = pl.program_id(0)
    g = _dot(fb_ref[...], fall_ref[...], ((1,), (1,)),
             prec=lax.Precision.DEFAULT)
    d2 = sqb_ref[...] + sqr_ref[...] - 2.0 * g
    iota_i = lax.broadcasted_iota(jnp.int32, (BLK_B, N), 1)
    iota_f = iota_i.astype(jnp.float32)
    rowid = i * BLK_B + lax.broadcasted_iota(jnp.int32, (BLK_B, N), 0)

    dpos = jnp.where(iota_i == rowid, BIG, d2)
    pos_i = []
    for _ in range(5):
        m = jnp.min(dpos, axis=1, keepdims=True)
        idx = jnp.min(jnp.where(dpos == m, iota_f, BIG), axis=1, keepdims=True)
        pos_i.append(idx)
        dpos = jnp.where(iota_f == idx, BIG, dpos)

    dneg = d2
    neg_i = [None] * 5
    for k in range(4, -1, -1):
        m = jnp.max(dneg, axis=1, keepdims=True)
        idx = jnp.max(jnp.where(dneg == m, iota_f, -1.0), axis=1, keepdims=True)
        neg_i[k] = idx
        dneg = jnp.where(iota_f == idx, -BIG, dneg)

    gemb = _dot(eb_ref[...], eall_ref[...], ((1,), (1,)))
    t = esqr_ref[...] - 2.0 * gemb

    def pick(vals):
        return [jnp.sum(jnp.where(iota_f == c, t, 0.0), axis=1, keepdims=True)
                for c in vals]

    d_pos = pick(pos_i)
    d_neg = pick(neg_i)

    best_v, best_i = d_pos[0], pos_i[0]
    for k in range(1, 5):
        upd = d_pos[k] > best_v
        best_v = jnp.where(upd, d_pos[k], best_v)
        best_i = jnp.where(upd, pos_i[k], best_i)
    pos_ref[...] = best_i.astype(jnp.int32)

    best_v, best_i = d_neg[0], neg_i[0]
    for k in range(1, 5):
        upd = d_neg[k] < best_v
        best_v = jnp.where(upd, d_neg[k], best_v)
        best_i = jnp.where(upd, neg_i[k], best_i)
    neg_ref[...] = best_i.astype(jnp.int32)


def _mine_call(feats, emb, sq, sq_c, esq_c):
    return pl.pallas_call(
        _mine_body,
        grid=(N // BLK_B,),
        in_specs=[
            pl.BlockSpec((BLK_B, F_IN), lambda i: (i, 0)),
            pl.BlockSpec((N, F_IN), lambda i: (0, 0)),
            pl.BlockSpec((BLK_B, D_EMB), lambda i: (i, 0)),
            pl.BlockSpec((N, D_EMB), lambda i: (0, 0)),
            pl.BlockSpec((BLK_B, 1), lambda i: (i, 0)),
            pl.BlockSpec((1, N), lambda i: (0, 0)),
            pl.BlockSpec((1, N), lambda i: (0, 0)),
        ],
        out_specs=[
            pl.BlockSpec((BLK_B, 1), lambda i: (i, 0)),
            pl.BlockSpec((BLK_B, 1), lambda i: (i, 0)),
        ],
        out_shape=[
            jax.ShapeDtypeStruct((N, 1), jnp.int32),
            jax.ShapeDtypeStruct((N, 1), jnp.int32),
        ],
    )(feats, feats, emb, emb, sq, sq_c, esq_c)


def _mesh_body(agg_ref, emb_ref, wm_ref, wc_ref, mg_ref, log_ref):
    aug = agg_ref[...]
    lane = lax.broadcasted_iota(jnp.int32, (N, AGG_W), 1)
    deg = jnp.sum(jnp.where(lane == D_EMB, aug, 0.0), axis=1, keepdims=True)
    agg = aug[:, :D_EMB] / jnp.maximum(deg, 1.0)
    h = jnp.maximum(_dot(agg, wm_ref[...], ((1,), (0,))), 0.0) + emb_ref[...]
    mg = jnp.sum(h, axis=0, keepdims=True) * (1.0 / N)
    mg_ref[...] = mg
    log_ref[...] = _dot(mg, wc_ref[...], ((1,), (0,)))


def _mesh_call(agg_aug, emb, w_mesh, w_cls):
    return pl.pallas_call(
        _mesh_body,
        out_shape=[
            jax.ShapeDtypeStruct((1, D_EMB), jnp.float32),
            jax.ShapeDtypeStruct((1, OUT_F), jnp.float32),
        ],
    )(agg_aug, emb, w_mesh, w_cls)


PASS_ROWS = 64


def _segsum_body(src_hbm, dst_hbm, aug_hbm, out_hbm,
                 srcb, dstb, slistA, dlistA, slistB, dlistB, agg, stage, sem):
    wid = lax.axis_index("s") * SC_NC + lax.axis_index("c")
    iota16 = lax.iota(jnp.int32, 16)
    zvec16 = jnp.zeros((16,), jnp.int32)
    lo = wid * ROWS_PER_TILE
    mid = lo + PASS_ROWS

    def scan_chunk(ci, offs):
        pltpu.sync_copy(src_hbm.at[pl.ds(ci * CHUNK, CHUNK)], srcb)
        pltpu.sync_copy(dst_hbm.at[pl.ds(ci * CHUNK, CHUNK)], dstb)

        def scan_vec(vi, offs2):
            off_a, off_b = offs2
            d = dstb[pl.ds(vi * 16, 16)]
            s = srcb[pl.ds(vi * 16, 16)]
            in_a = (d >= lo) & (d < mid)
            in_b = (d >= mid) & (d < lo + ROWS_PER_TILE)
            cum_a = plsc.cumsum(jnp.where(in_a, 1, 0))
            cum_b = plsc.cumsum(jnp.where(in_b, 1, 0))
            pos_a = jnp.where(in_a, off_a + cum_a - 1, CAP + 8)
            pos_b = jnp.where(in_b, off_b + cum_b - 1, CAP + 8)
            plsc.store_scatter(slistA, [pos_a], s)
            plsc.store_scatter(dlistA, [pos_a >> 4, pos_a & 15], d - lo)
            plsc.store_scatter(slistB, [pos_b], s)
            plsc.store_scatter(dlistB, [pos_b >> 4, pos_b & 15], d - mid)
            return (off_a + cum_a[15], off_b + cum_b[15])
        return lax.fori_loop(0, CHUNK // 16, scan_vec, offs)
    off_a, off_b = lax.fori_loop(0, E_EDGES // CHUNK, scan_chunk,
                                 (jnp.int32(0), jnp.int32(0)))

    for slist, dlist, off, base in ((slistA, dlistA, off_a, lo),
                                    (slistB, dlistB, off_b, mid)):
        pos_pad = off + iota16
        plsc.store_scatter(slist, [pos_pad], zvec16)
        plsc.store_scatter(dlist, [pos_pad >> 4, pos_pad & 15],
                           jnp.full((16,), PASS_ROWS, jnp.int32))

        def zero_row(r, carry):
            def zero_col(c, carry2):
                agg[r, pl.ds(c * 16, 16)] = jnp.zeros((16,), jnp.float32)
                return carry2
            return lax.fori_loop(0, AGG_W // 16, zero_col, carry)
        lax.fori_loop(0, PASS_ROWS + 4, zero_row, 0)

        nb = (off + 15) >> 4

        def start_gather(b):
            sidx = slist[pl.ds(b * 16, 16)]
            pltpu.async_copy(aug_hbm.at[sidx], stage.at[b % 3], sem)

        @pl.when(nb > 0)
        def _():
            start_gather(jnp.int32(0))

        @pl.when(nb > 1)
        def _():
            start_gather(jnp.int32(1))

        def proc(bi, carry):
            cur = bi % 3
            pltpu.make_async_copy(aug_hbm.at[pl.ds(0, 16)],
                                  stage.at[cur], sem).wait()

            @pl.when(bi + 2 < nb)
            def _():
                start_gather(bi + 2)
            for r in range(16):
                row_idx = plsc.load_gather(dlist, [bi + zvec16, r + zvec16])
                for c in range(AGG_W // 16):
                    plsc.addupdate_scatter(
                        agg, [row_idx, c * 16 + iota16],
                        stage[cur, r, pl.ds(c * 16, 16)])
            return carry
        lax.fori_loop(0, nb, proc, 0)

        pltpu.sync_copy(agg.at[pl.ds(0, PASS_ROWS)],
                        out_hbm.at[pl.ds(base, PASS_ROWS)])


def _segsum_call(src, dst, emb_aug):
    mesh = plsc.VectorSubcoreMesh(core_axis_name="c", subcore_axis_name="s")
    k = functools.partial(
        pl.kernel,
        mesh=mesh,
        compiler_params=pltpu.CompilerParams(needs_layout_passes=False),
        out_type=jax.ShapeDtypeStruct((N, AGG_W), jnp.float32),
        scratch_types=[
            pltpu.VMEM((CHUNK,), jnp.int32),
            pltpu.VMEM((CHUNK,), jnp.int32),
            pltpu.VMEM((CAP + 16,), jnp.int32),
            pltpu.VMEM((CAP // 16 + 2, 16), jnp.int32),
            pltpu.VMEM((CAP + 16,), jnp.int32),
            pltpu.VMEM((CAP // 16 + 2, 16), jnp.int32),
            pltpu.VMEM((PASS_ROWS + 4, AGG_W), jnp.float32),
            pltpu.VMEM((3, 16, AUG), jnp.float32),
            pltpu.SemaphoreType.DMA,
        ],
    )(_segsum_body)
    return k(src, dst, emb_aug)


def _gather_body(emb_hbm, pos_hbm, neg_hbm, pos_out, neg_out,
                 idx_v, rows_v, sem):
    wid = lax.axis_index("s") * SC_NC + lax.axis_index("c")
    base = wid * GATHER_B
    pltpu.sync_copy(pos_hbm.at[pl.ds(base, GATHER_B)], idx_v)
    pltpu.async_copy(emb_hbm.at[idx_v], rows_v, sem).wait()
    pltpu.sync_copy(rows_v, pos_out.at[pl.ds(base, GATHER_B)])
    pltpu.sync_copy(neg_hbm.at[pl.ds(base, GATHER_B)], idx_v)
    pltpu.async_copy(emb_hbm.at[idx_v], rows_v, sem).wait()
    pltpu.sync_copy(rows_v, neg_out.at[pl.ds(base, GATHER_B)])


def _gather_call(emb, pos_idx, neg_idx):
    mesh = plsc.VectorSubcoreMesh(core_axis_name="c", subcore_axis_name="s")
    k = functools.partial(
        pl.kernel,
        mesh=mesh,
        compiler_params=pltpu.CompilerParams(needs_layout_passes=False),
        out_type=(
            jax.ShapeDtypeStruct((N, D_EMB), jnp.float32),
            jax.ShapeDtypeStruct((N, D_EMB), jnp.float32),
        ),
        scratch_types=[
            pltpu.VMEM((GATHER_B,), jnp.int32),
            pltpu.VMEM((GATHER_B, D_EMB), jnp.float32),
            pltpu.SemaphoreType.DMA,
        ],
    )(_gather_body)
    return k(emb, pos_idx, neg_idx)


def kernel(feats, edge_index, W_patch, W_mesh, W_cls):
    emb, emb_aug, sq, esq = _embed_call(feats, W_patch)
    sq_c = sq.reshape(1, N)
    esq_c = esq.reshape(1, N)
    agg_aug = _segsum_call(edge_index[0], edge_index[1], emb_aug)
    pos_i, neg_i = _mine_call(feats, emb, sq, sq_c, esq_c)
    mg, logits = _mesh_call(agg_aug, emb, W_mesh, W_cls)
    pos_emb, neg_emb = _gather_call(emb, pos_i.reshape(N), neg_i.reshape(N))
    return (logits.reshape(OUT_F), mg.reshape(D_EMB), emb, pos_emb, neg_emb)


# speedup vs baseline: 1.2698x; 1.0030x over previous
import functools

import jax
import jax.numpy as jnp
from jax import lax
from jax.experimental import pallas as pl
from jax.experimental.pallas import tpu as pltpu
from jax.experimental.pallas import tpu_sc as plsc

N = 4096
F_IN = 512
D_EMB = 512
OUT_F = 128
E_EDGES = 65536
AUG = 640
AGG_W = 528
BLK_A = 256
BLK_B = 128
BIG = 1.0e30
PREC = lax.Precision.HIGHEST

SC_NC = 2
SC_NS = 16
NW = SC_NC * SC_NS
ROWS_PER_TILE = N // NW
CAP = 2048
CHUNK = 2048
GATHER_B = N // NW


def _dot(a, b, dims, prec=PREC):
    return lax.dot_general(a, b, (dims, ((), ())),
                           preferred_element_type=jnp.float32, precision=prec)


def _embed_body(feats_ref, wp_ref, emb_ref, aug_ref, sq_ref, esq_ref):
    f = feats_ref[...]
    e = jnp.maximum(_dot(f, wp_ref[...], ((1,), (0,)),
                         prec=lax.Precision.DEFAULT), 0.0)
    emb_ref[...] = e
    ones_col = (lax.broadcasted_iota(jnp.int32, (BLK_A, AUG - D_EMB), 1) == 0)
    aug_ref[...] = jnp.concatenate([e, ones_col.astype(jnp.float32)], axis=1)
    sq_ref[...] = jnp.sum(f * f, axis=1, keepdims=True)
    esq_ref[...] = jnp.sum(e * e, axis=1, keepdims=True)


def _embed_call(feats, w_patch):
    return pl.pallas_call(
        _embed_body,
        grid=(N // BLK_A,),
        in_specs=[
            pl.BlockSpec((BLK_A, F_IN), lambda i: (i, 0)),
            pl.BlockSpec((F_IN, D_EMB), lambda i: (0, 0)),
        ],
        out_specs=[
            pl.BlockSpec((BLK_A, D_EMB), lambda i: (i, 0)),
            pl.BlockSpec((BLK_A, AUG), lambda i: (i, 0)),
            pl.BlockSpec((BLK_A, 1), lambda i: (i, 0)),
            pl.BlockSpec((BLK_A, 1), lambda i: (i, 0)),
        ],
        out_shape=[
            jax.ShapeDtypeStruct((N, D_EMB), jnp.float32),
            jax.ShapeDtypeStruct((N, AUG), jnp.float32),
            jax.ShapeDtypeStruct((N, 1), jnp.float32),
            jax.ShapeDtypeStruct((N, 1), jnp.float32),
        ],
    )(feats, w_patch)


def _mine_body(fb_ref, fall_ref, eb_ref, eall_ref, sqb_ref, sqr_ref, esqr_ref,
               pos_ref, neg_ref):
    i = pl.program_id(0)
    g = _dot(fb_ref[...], fall_ref[...], ((1,), (1,)),
             prec=lax.Precision.DEFAULT)
    d2 = sqb_ref[...] + sqr_ref[...] - 2.0 * g
    iota_i = lax.broadcasted_iota(jnp.int32, (BLK_B, N), 1)
    iota_f = iota_i.astype(jnp.float32)
    rowid = i * BLK_B + lax.broadcasted_iota(jnp.int32, (BLK_B, N), 0)

    dpos = jnp.where(iota_i == rowid, BIG, d2)
    pos_i = []
    for _ in range(5):
        m = jnp.min(dpos, axis=1, keepdims=True)
        idx = jnp.min(jnp.where(dpos == m, iota_f, BIG), axis=1, keepdims=True)
        pos_i.append(idx)
        dpos = jnp.where(iota_f == idx, BIG, dpos)

    dneg = d2
    neg_i = [None] * 5
    for k in range(4, -1, -1):
        m = jnp.max(dneg, axis=1, keepdims=True)
        idx = jnp.max(jnp.where(dneg == m, iota_f, -1.0), axis=1, keepdims=True)
        neg_i[k] = idx
        dneg = jnp.where(iota_f == idx, -BIG, dneg)

    gemb = _dot(eb_ref[...], eall_ref[...], ((1,), (1,)))
    t = esqr_ref[...] - 2.0 * gemb

    def pick(vals):
        return [jnp.sum(jnp.where(iota_f == c, t, 0.0), axis=1, keepdims=True)
                for c in vals]

    d_pos = pick(pos_i)
    d_neg = pick(neg_i)

    best_v, best_i = d_pos[0], pos_i[0]
    for k in range(1, 5):
        upd = d_pos[k] > best_v
        best_v = jnp.where(upd, d_pos[k], best_v)
        best_i = jnp.where(upd, pos_i[k], best_i)
    pos_ref[...] = best_i.astype(jnp.int32)

    best_v, best_i = d_neg[0], neg_i[0]
    for k in range(1, 5):
        upd = d_neg[k] < best_v
        best_v = jnp.where(upd, d_neg[k], best_v)
        best_i = jnp.where(upd, neg_i[k], best_i)
    neg_ref[...] = best_i.astype(jnp.int32)


def _mine_call(feats, emb, sq, sq_c, esq_c):
    return pl.pallas_call(
        _mine_body,
        grid=(N // BLK_B,),
        in_specs=[
            pl.BlockSpec((BLK_B, F_IN), lambda i: (i, 0)),
            pl.BlockSpec((N, F_IN), lambda i: (0, 0)),
            pl.BlockSpec((BLK_B, D_EMB), lambda i: (i, 0)),
            pl.BlockSpec((N, D_EMB), lambda i: (0, 0)),
            pl.BlockSpec((BLK_B, 1), lambda i: (i, 0)),
            pl.BlockSpec((1, N), lambda i: (0, 0)),
            pl.BlockSpec((1, N), lambda i: (0, 0)),
        ],
        out_specs=[
            pl.BlockSpec((BLK_B, 1), lambda i: (i, 0)),
            pl.BlockSpec((BLK_B, 1), lambda i: (i, 0)),
        ],
        out_shape=[
            jax.ShapeDtypeStruct((N, 1), jnp.int32),
            jax.ShapeDtypeStruct((N, 1), jnp.int32),
        ],
    )(feats, feats, emb, emb, sq, sq_c, esq_c)


def _mesh_body(agg_ref, emb_ref, wm_ref, wc_ref, mg_ref, log_ref):
    aug = agg_ref[...]
    lane = lax.broadcasted_iota(jnp.int32, (N, AGG_W), 1)
    deg = jnp.sum(jnp.where(lane == D_EMB, aug, 0.0), axis=1, keepdims=True)
    agg = aug[:, :D_EMB] / jnp.maximum(deg, 1.0)
    h = jnp.maximum(_dot(agg, wm_ref[...], ((1,), (0,))), 0.0) + emb_ref[...]
    mg = jnp.sum(h, axis=0, keepdims=True) * (1.0 / N)
    mg_ref[...] = mg
    log_ref[...] = _dot(mg, wc_ref[...], ((1,), (0,)))


def _mesh_call(agg_aug, emb, w_mesh, w_cls):
    return pl.pallas_call(
        _mesh_body,
        out_shape=[
            jax.ShapeDtypeStruct((1, D_EMB), jnp.float32),
            jax.ShapeDtypeStruct((1, OUT_F), jnp.float32),
        ],
    )(agg_aug, emb, w_mesh, w_cls)


PASS_ROWS = 64


def _segsum_body(src_hbm, dst_hbm, aug_hbm, out_hbm,
                 srcb, dstb, slist, dlist, agg, stage, sem):
    wid = lax.axis_index("s") * SC_NC + lax.axis_index("c")
    iota16 = lax.iota(jnp.int32, 16)
    zvec16 = jnp.zeros((16,), jnp.int32)

    for p in range(ROWS_PER_TILE // PASS_ROWS):
        lo = wid * ROWS_PER_TILE + p * PASS_ROWS

        def zero_row(r, carry):
            def zero_col(c, carry2):
                agg[r, pl.ds(c * 16, 16)] = jnp.zeros((16,), jnp.float32)
                return carry2
            return lax.fori_loop(0, AGG_W // 16, zero_col, carry)
        lax.fori_loop(0, PASS_ROWS + 4, zero_row, 0)

        def scan_chunk(ci, off):
            pltpu.sync_copy(src_hbm.at[pl.ds(ci * CHUNK, CHUNK)], srcb)
            pltpu.sync_copy(dst_hbm.at[pl.ds(ci * CHUNK, CHUNK)], dstb)

            def scan_vec(vi, off2):
                d = dstb[pl.ds(vi * 16, 16)]
                s = srcb[pl.ds(vi * 16, 16)]
                msk = (d >= lo) & (d < lo + PASS_ROWS)
                mi = jnp.where(msk, 1, 0)
                cum = plsc.cumsum(mi)
                pos = jnp.where(msk, off2 + cum - 1, CAP + 8)
                plsc.store_scatter(slist, [pos], s)
                plsc.store_scatter(dlist, [pos >> 4, pos & 15], d - lo)
                return off2 + cum[15]
            return lax.fori_loop(0, CHUNK // 16, scan_vec, off)
        off = lax.fori_loop(0, E_EDGES // CHUNK, scan_chunk, jnp.int32(0))

        pos_pad = off + iota16
        plsc.store_scatter(slist, [pos_pad], zvec16)
        plsc.store_scatter(dlist, [pos_pad >> 4, pos_pad & 15],
                           jnp.full((16,), PASS_ROWS, jnp.int32))

        nb = (off + 15) >> 4

        def start_gather(b):
            sidx = slist[pl.ds(b * 16, 16)]
            pltpu.async_copy(aug_hbm.at[sidx], stage.at[b % 3], sem)

        @pl.when(nb > 0)
        def _():
            start_gather(jnp.int32(0))

        @pl.when(nb > 1)
        def _():
            start_gather(jnp.int32(1))

        def proc(bi, carry):
            cur = bi % 3
            pltpu.make_async_copy(aug_hbm.at[pl.ds(0, 16)],
                                  stage.at[cur], sem).wait()

            @pl.when(bi + 2 < nb)
            def _():
                start_gather(bi + 2)

            @plsc.parallel_loop(0, 16, step=1, carry=jnp.int32(0))
            def add_row(r, j):
                row_idx = plsc.load_gather(dlist, [bi + zvec16, r + zvec16])
                for c in range(AGG_W // 16):
                    plsc.addupdate_scatter(
                        agg, [row_idx, c * 16 + iota16],
                        stage[cur, r, pl.ds(c * 16, 16)])
                return j
            return carry
        lax.fori_loop(0, nb, proc, 0)

        pltpu.sync_copy(agg.at[pl.ds(0, PASS_ROWS)],
                        out_hbm.at[pl.ds(lo, PASS_ROWS)])


def _segsum_call(src, dst, emb_aug):
    mesh = plsc.VectorSubcoreMesh(core_axis_name="c", subcore_axis_name="s")
    k = functools.partial(
        pl.kernel,
        mesh=mesh,
        compiler_params=pltpu.CompilerParams(needs_layout_passes=False),
        out_type=jax.ShapeDtypeStruct((N, AGG_W), jnp.float32),
        scratch_types=[
            pltpu.VMEM((CHUNK,), jnp.int32),
            pltpu.VMEM((CHUNK,), jnp.int32),
            pltpu.VMEM((CAP + 16,), jnp.int32),
            pltpu.VMEM((CAP // 16 + 2, 16), jnp.int32),
            pltpu.VMEM((PASS_ROWS + 4, AGG_W), jnp.float32),
            pltpu.VMEM((3, 16, AUG), jnp.float32),
            pltpu.SemaphoreType.DMA,
        ],
    )(_segsum_body)
    return k(src, dst, emb_aug)


def _gather_body(emb_hbm, pos_hbm, neg_hbm, pos_out, neg_out,
                 idx_v, rows_v, sem):
    wid = lax.axis_index("s") * SC_NC + lax.axis_index("c")
    base = wid * GATHER_B
    pltpu.sync_copy(pos_hbm.at[pl.ds(base, GATHER_B)], idx_v)
    pltpu.async_copy(emb_hbm.at[idx_v], rows_v, sem).wait()
    pltpu.sync_copy(rows_v, pos_out.at[pl.ds(base, GATHER_B)])
    pltpu.sync_copy(neg_hbm.at[pl.ds(base, GATHER_B)], idx_v)
    pltpu.async_copy(emb_hbm.at[idx_v], rows_v, sem).wait()
    pltpu.sync_copy(rows_v, neg_out.at[pl.ds(base, GATHER_B)])


def _gather_call(emb, pos_idx, neg_idx):
    mesh = plsc.VectorSubcoreMesh(core_axis_name="c", subcore_axis_name="s")
    k = functools.partial(
        pl.kernel,
        mesh=mesh,
        compiler_params=pltpu.CompilerParams(needs_layout_passes=False),
        out_type=(
            jax.ShapeDtypeStruct((N, D_EMB), jnp.float32),
            jax.ShapeDtypeStruct((N, D_EMB), jnp.float32),
        ),
        scratch_types=[
            pltpu.VMEM((GATHER_B,), jnp.int32),
            pltpu.VMEM((GATHER_B, D_EMB), jnp.float32),
            pltpu.SemaphoreType.DMA,
        ],
    )(_gather_body)
    return k(emb, pos_idx, neg_idx)


def kernel(feats, edge_index, W_patch, W_mesh, W_cls):
    emb, emb_aug, sq, esq = _embed_call(feats, W_patch)
    sq_c = sq.reshape(1, N)
    esq_c = esq.reshape(1, N)
    agg_aug = _segsum_call(edge_index[0], edge_index[1], emb_aug)
    pos_i, neg_i = _mine_call(feats, emb, sq, sq_c, esq_c)
    mg, logits = _mesh_call(agg_aug, emb, W_mesh, W_cls)
    pos_emb, neg_emb = _gather_call(emb, pos_i.reshape(N), neg_i.reshape(N))
    return (logits.reshape(OUT_F), mg.reshape(D_EMB), emb, pos_emb, neg_emb)


# speedup vs baseline: 1.2816x; 1.0093x over previous
import functools

import jax
import jax.numpy as jnp
from jax import lax
from jax.experimental import pallas as pl
from jax.experimental.pallas import tpu as pltpu
from jax.experimental.pallas import tpu_sc as plsc

N = 4096
F_IN = 512
D_EMB = 512
OUT_F = 128
E_EDGES = 65536
AUG = 640
AGG_W = 528
BLK_A = 256
BLK_B = 128
BIG = 1.0e30
PREC = lax.Precision.HIGHEST

SC_NC = 2
SC_NS = 16
NW = SC_NC * SC_NS
ROWS_PER_TILE = N // NW
CAP = 2048
CHUNK = 2048
GATHER_B = N // NW


def _dot(a, b, dims, prec=PREC):
    return lax.dot_general(a, b, (dims, ((), ())),
                           preferred_element_type=jnp.float32, precision=prec)


def _embed_body(feats_ref, wp_ref, emb_ref, aug_ref, sq_ref, esq_ref):
    f = feats_ref[...]
    e = jnp.maximum(_dot(f, wp_ref[...], ((1,), (0,)),
                         prec=lax.Precision.DEFAULT), 0.0)
    emb_ref[...] = e
    ones_col = (lax.broadcasted_iota(jnp.int32, (BLK_A, AUG - D_EMB), 1) == 0)
    aug_ref[...] = jnp.concatenate([e, ones_col.astype(jnp.float32)], axis=1)
    sq_ref[...] = jnp.sum(f * f, axis=1, keepdims=True)
    esq_ref[...] = jnp.sum(e * e, axis=1, keepdims=True)


def _embed_call(feats, w_patch):
    return pl.pallas_call(
        _embed_body,
        grid=(N // BLK_A,),
        in_specs=[
            pl.BlockSpec((BLK_A, F_IN), lambda i: (i, 0)),
            pl.BlockSpec((F_IN, D_EMB), lambda i: (0, 0)),
        ],
        out_specs=[
            pl.BlockSpec((BLK_A, D_EMB), lambda i: (i, 0)),
            pl.BlockSpec((BLK_A, AUG), lambda i: (i, 0)),
            pl.BlockSpec((BLK_A, 1), lambda i: (i, 0)),
            pl.BlockSpec((BLK_A, 1), lambda i: (i, 0)),
        ],
        out_shape=[
            jax.ShapeDtypeStruct((N, D_EMB), jnp.float32),
            jax.ShapeDtypeStruct((N, AUG), jnp.float32),
            jax.ShapeDtypeStruct((N, 1), jnp.float32),
            jax.ShapeDtypeStruct((N, 1), jnp.float32),
        ],
    )(feats, w_patch)


def _mine_body(fb_ref, fall_ref, eb_ref, eall_ref, sqb_ref, sqr_ref, esqr_ref,
               pos_ref, neg_ref):
    i = pl.program_id(0)
    g = _dot(fb_ref[...], fall_ref[...], ((1,), (1,)),
             prec=lax.Precision.DEFAULT)
    d2 = sqb_ref[...] + sqr_ref[...] - 2.0 * g
    iota_i = lax.broadcasted_iota(jnp.int32, (BLK_B, N), 1)
    iota_f = iota_i.astype(jnp.float32)
    rowid = i * BLK_B + lax.broadcasted_iota(jnp.int32, (BLK_B, N), 0)

    dpos = jnp.where(iota_i == rowid, BIG, d2)
    pos_i = []
    for _ in range(5):
        m = jnp.min(dpos, axis=1, keepdims=True)
        idx = jnp.min(jnp.where(dpos == m, iota_f, BIG), axis=1, keepdims=True)
        pos_i.append(idx)
        dpos = jnp.where(iota_f == idx, BIG, dpos)

    dneg = d2
    neg_i = [None] * 5
    for k in range(4, -1, -1):
        m = jnp.max(dneg, axis=1, keepdims=True)
        idx = jnp.max(jnp.where(dneg == m, iota_f, -1.0), axis=1, keepdims=True)
        neg_i[k] = idx
        dneg = jnp.where(iota_f == idx, -BIG, dneg)

    gemb = _dot(eb_ref[...], eall_ref[...], ((1,), (1,)))
    t = esqr_ref[...] - 2.0 * gemb

    def pick(vals):
        return [jnp.sum(jnp.where(iota_f == c, t, 0.0), axis=1, keepdims=True)
                for c in vals]

    d_pos = pick(pos_i)
    d_neg = pick(neg_i)

    best_v, best_i = d_pos[0], pos_i[0]
    for k in range(1, 5):
        upd = d_pos[k] > best_v
        best_v = jnp.where(upd, d_pos[k], best_v)
        best_i = jnp.where(upd, pos_i[k], best_i)
    pos_ref[...] = best_i.astype(jnp.int32)

    best_v, best_i = d_neg[0], neg_i[0]
    for k in range(1, 5):
        upd = d_neg[k] < best_v
        best_v = jnp.where(upd, d_neg[k], best_v)
        best_i = jnp.where(upd, neg_i[k], best_i)
    neg_ref[...] = best_i.astype(jnp.int32)


def _mine_call(feats, emb, sq, sq_c, esq_c):
    return pl.pallas_call(
        _mine_body,
        grid=(N // BLK_B,),
        in_specs=[
            pl.BlockSpec((BLK_B, F_IN), lambda i: (i, 0)),
            pl.BlockSpec((N, F_IN), lambda i: (0, 0)),
            pl.BlockSpec((BLK_B, D_EMB), lambda i: (i, 0)),
            pl.BlockSpec((N, D_EMB), lambda i: (0, 0)),
            pl.BlockSpec((BLK_B, 1), lambda i: (i, 0)),
            pl.BlockSpec((1, N), lambda i: (0, 0)),
            pl.BlockSpec((1, N), lambda i: (0, 0)),
        ],
        out_specs=[
            pl.BlockSpec((BLK_B, 1), lambda i: (i, 0)),
            pl.BlockSpec((BLK_B, 1), lambda i: (i, 0)),
        ],
        out_shape=[
            jax.ShapeDtypeStruct((N, 1), jnp.int32),
            jax.ShapeDtypeStruct((N, 1), jnp.int32),
        ],
    )(feats, feats, emb, emb, sq, sq_c, esq_c)


def _mesh_body(agg_ref, emb_ref, wm_ref, wc_ref, mg_ref, log_ref):
    aug = agg_ref[...]
    lane = lax.broadcasted_iota(jnp.int32, (N, AGG_W), 1)
    deg = jnp.sum(jnp.where(lane == D_EMB, aug, 0.0), axis=1, keepdims=True)
    agg = aug[:, :D_EMB] / jnp.maximum(deg, 1.0)
    h = jnp.maximum(_dot(agg, wm_ref[...], ((1,), (0,))), 0.0) + emb_ref[...]
    mg = jnp.sum(h, axis=0, keepdims=True) * (1.0 / N)
    mg_ref[...] = mg
    log_ref[...] = _dot(mg, wc_ref[...], ((1,), (0,)))


def _mesh_call(agg_aug, emb, w_mesh, w_cls):
    return pl.pallas_call(
        _mesh_body,
        out_shape=[
            jax.ShapeDtypeStruct((1, D_EMB), jnp.float32),
            jax.ShapeDtypeStruct((1, OUT_F), jnp.float32),
        ],
    )(agg_aug, emb, w_mesh, w_cls)


PASS_ROWS = 64


def _segsum_body(src_hbm, dst_hbm, aug_hbm, out_hbm,
                 srcb, dstb, slist, dlist, agg, stage, sem):
    wid = lax.axis_index("s") * SC_NC + lax.axis_index("c")
    iota16 = lax.iota(jnp.int32, 16)
    zvec16 = jnp.zeros((16,), jnp.int32)

    for p in range(ROWS_PER_TILE // PASS_ROWS):
        lo = wid * ROWS_PER_TILE + p * PASS_ROWS

        def zero_row(r, carry):
            def zero_col(c, carry2):
                agg[r, pl.ds(c * 16, 16)] = jnp.zeros((16,), jnp.float32)
                return carry2
            return lax.fori_loop(0, AGG_W // 16, zero_col, carry)
        lax.fori_loop(0, PASS_ROWS + 4, zero_row, 0)

        def scan_chunk(ci, off):
            pltpu.sync_copy(src_hbm.at[pl.ds(ci * CHUNK, CHUNK)], srcb)
            pltpu.sync_copy(dst_hbm.at[pl.ds(ci * CHUNK, CHUNK)], dstb)

            def scan_vec(vi, off2):
                d = dstb[pl.ds(vi * 16, 16)]
                s = srcb[pl.ds(vi * 16, 16)]
                msk = (d >= lo) & (d < lo + PASS_ROWS)
                mi = jnp.where(msk, 1, 0)
                cum = plsc.cumsum(mi)
                pos = jnp.where(msk, off2 + cum - 1, CAP + 8)
                plsc.store_scatter(slist, [pos], s)
                plsc.store_scatter(dlist, [pos >> 4, pos & 15], d - lo)
                return off2 + cum[15]
            return lax.fori_loop(0, CHUNK // 16, scan_vec, off)
        off = lax.fori_loop(0, E_EDGES // CHUNK, scan_chunk, jnp.int32(0))

        pos_pad = off + iota16
        plsc.store_scatter(slist, [pos_pad], zvec16)
        plsc.store_scatter(dlist, [pos_pad >> 4, pos_pad & 15],
                           jnp.full((16,), PASS_ROWS, jnp.int32))

        nb = (off + 15) >> 4

        def start_gather(b):
            sidx = slist[pl.ds(b * 16, 16)]
            pltpu.async_copy(aug_hbm.at[sidx], stage.at[b % 3], sem)

        @pl.when(nb > 0)
        def _():
            start_gather(jnp.int32(0))

        @pl.when(nb > 1)
        def _():
            start_gather(jnp.int32(1))

        def proc(bi, carry):
            cur = bi % 3
            pltpu.make_async_copy(aug_hbm.at[pl.ds(0, 16)],
                                  stage.at[cur], sem).wait()

            @pl.when(bi + 2 < nb)
            def _():
                start_gather(bi + 2)
            for r in range(16):
                row_idx = plsc.load_gather(dlist, [bi + zvec16, r + zvec16])
                for c in range(AGG_W // 16):
                    plsc.addupdate_scatter(
                        agg, [row_idx, c * 16 + iota16],
                        stage[cur, r, pl.ds(c * 16, 16)])
            return carry
        lax.fori_loop(0, nb, proc, 0)

        pltpu.sync_copy(agg.at[pl.ds(0, PASS_ROWS)],
                        out_hbm.at[pl.ds(lo, PASS_ROWS)])


def _segsum_call(src, dst, emb_aug):
    mesh = plsc.VectorSubcoreMesh(core_axis_name="c", subcore_axis_name="s")
    k = functools.partial(
        pl.kernel,
        mesh=mesh,
        compiler_params=pltpu.CompilerParams(needs_layout_passes=False),
        out_type=jax.ShapeDtypeStruct((N, AGG_W), jnp.float32),
        scratch_types=[
            pltpu.VMEM((CHUNK,), jnp.int32),
            pltpu.VMEM((CHUNK,), jnp.int32),
            pltpu.VMEM((CAP + 16,), jnp.int32),
            pltpu.VMEM((CAP // 16 + 2, 16), jnp.int32),
            pltpu.VMEM((PASS_ROWS + 4, AGG_W), jnp.float32),
            pltpu.VMEM((3, 16, AUG), jnp.float32),
            pltpu.SemaphoreType.DMA,
        ],
    )(_segsum_body)
    return k(src, dst, emb_aug)


def _gather_body(emb_hbm, pos_hbm, neg_hbm, pos_out, neg_out,
                 idx_v, rows_v, sem):
    wid = lax.axis_index("s") * SC_NC + lax.axis_index("c")
    base = wid * GATHER_B
    pltpu.sync_copy(pos_hbm.at[pl.ds(base, GATHER_B)], idx_v)
    pltpu.async_copy(emb_hbm.at[idx_v], rows_v, sem).wait()
    pltpu.sync_copy(rows_v, pos_out.at[pl.ds(base, GATHER_B)])
    pltpu.sync_copy(neg_hbm.at[pl.ds(base, GATHER_B)], idx_v)
    pltpu.async_copy(emb_hbm.at[idx_v], rows_v, sem).wait()
    pltpu.sync_copy(rows_v, neg_out.at[pl.ds(base, GATHER_B)])


def _gather_call(emb, pos_idx, neg_idx):
    mesh = plsc.VectorSubcoreMesh(core_axis_name="c", subcore_axis_name="s")
    k = functools.partial(
        pl.kernel,
        mesh=mesh,
        compiler_params=pltpu.CompilerParams(needs_layout_passes=False),
        out_type=(
            jax.ShapeDtypeStruct((N, D_EMB), jnp.float32),
            jax.ShapeDtypeStruct((N, D_EMB), jnp.float32),
        ),
        scratch_types=[
            pltpu.VMEM((GATHER_B,), jnp.int32),
            pltpu.VMEM((GATHER_B, D_EMB), jnp.float32),
            pltpu.SemaphoreType.DMA,
        ],
    )(_gather_body)
    return k(emb, pos_idx, neg_idx)


def kernel(feats, edge_index, W_patch, W_mesh, W_cls):
    emb, emb_aug, sq, esq = _embed_call(feats, W_patch)
    sq_c = sq.reshape(1, N)
    esq_c = esq.reshape(1, N)
    agg_aug = _segsum_call(edge_index[0], edge_index[1], emb_aug)
    pos_i, neg_i = _mine_call(feats, emb, sq, sq_c, esq_c)
    mg, logits = _mesh_call(agg_aug, emb, W_mesh, W_cls)
    pos_emb, neg_emb = _gather_call(emb, pos_i.reshape(N), neg_i.reshape(N))
    return (logits.reshape(OUT_F), mg.reshape(D_EMB), emb, pos_emb, neg_emb)
